# Initial kernel scaffold; baseline (speedup 1.0000x reference)
import jax
import jax.numpy as jnp
from jax.experimental import pallas as pl


def kernel(xyz, points):
    raise NotImplementedError("write your pallas kernel here")


# speedup vs baseline: 1.8810x; 1.8810x over previous
import functools

import jax
import jax.numpy as jnp
from jax import lax
from jax.experimental import pallas as pl
from jax.experimental.pallas import tpu as pltpu
from jax.experimental.pallas import tpu_sc as plsc

NPOINT = 512
RADIUS = 0.2
NSAMPLE = 32
B, CIN, N = 8, 128, 4096
L = 16


def _fps_and_ballquery(xyz_t):
    Bb, Nn, C = xyz_t.shape
    batch_idx = jnp.arange(Bb)

    def step(carry, _):
        distance, farthest = carry
        centroid = xyz_t[batch_idx, farthest, :].reshape(Bb, 1, C)
        dist = jnp.sum((xyz_t - centroid) ** 2, -1)
        distance = jnp.where(dist < distance, dist, distance)
        new_farthest = jnp.argmax(distance, -1).astype(jnp.int32)
        return (distance, new_farthest), farthest

    distance0 = jnp.ones((Bb, Nn), dtype=xyz_t.dtype) * 1e10
    farthest0 = jnp.zeros((Bb,), dtype=jnp.int32)
    (_, _), centroids = lax.scan(step, (distance0, farthest0), None, length=NPOINT)
    fps_idx = jnp.transpose(centroids, (1, 0))

    new_xyz = xyz_t[batch_idx[:, None], fps_idx]
    sq = -2 * jnp.matmul(new_xyz, jnp.transpose(xyz_t, (0, 2, 1)))
    sq = sq + jnp.sum(new_xyz ** 2, -1).reshape(Bb, NPOINT, 1)
    sq = sq + jnp.sum(xyz_t ** 2, -1).reshape(Bb, 1, Nn)
    mask = ~(sq > RADIUS ** 2)
    cnt = jnp.cumsum(mask.astype(jnp.int32), axis=-1)
    cntc = jnp.minimum(cnt, NSAMPLE + 2)
    idx = jnp.stack([(cntc <= k).sum(-1) for k in range(NSAMPLE)], axis=-1)
    first = idx[:, :, 0:1]
    idx = jnp.where(idx >= Nn, first, idx)
    idx = jnp.minimum(idx, Nn - 1).astype(jnp.int32)
    return fps_idx, idx


_mesh = plsc.VectorSubcoreMesh(core_axis_name="c", subcore_axis_name="s")
_KS = NSAMPLE * NPOINT


@functools.partial(
    pl.kernel,
    mesh=_mesh,
    out_type=jax.ShapeDtypeStruct((B, 2 * CIN, _KS), jnp.float32),
    compiler_params=pltpu.CompilerParams(needs_layout_passes=False),
    scratch_types=[
        pltpu.VMEM((_KS,), jnp.int32),
        pltpu.VMEM((NPOINT,), jnp.int32),
        pltpu.VMEM((N,), jnp.float32),
        pltpu.VMEM((NPOINT,), jnp.float32),
        pltpu.VMEM((_KS,), jnp.float32),
        pltpu.VMEM((_KS,), jnp.float32),
    ],
)
def _sc_gather(points_hbm, idx_hbm, fps_hbm, out_hbm,
               idx_v, fps_v, tab_v, ctr_v, o1_v, o2_v):
    wid = lax.axis_index("s") * 2 + lax.axis_index("c")
    b = wid // 4
    g = wid % 4
    pltpu.sync_copy(idx_hbm.at[b], idx_v)
    pltpu.sync_copy(fps_hbm.at[b], fps_v)

    def chan_body(ci, _):
        c = g * (CIN // 4) + ci
        pltpu.sync_copy(points_hbm.at[b, c], tab_v)

        def ctr_body(i, _):
            iv = fps_v[pl.ds(i * L, L)]
            ctr_v[pl.ds(i * L, L)] = plsc.load_gather(tab_v, [iv])
            return 0

        lax.fori_loop(0, NPOINT // L, ctr_body, 0)

        def s_body(si, _):
            cv = ctr_v[pl.ds(si * L, L)]

            def k_body(k, _):
                off = k * NPOINT + si * L
                gv = plsc.load_gather(tab_v, [idx_v[pl.ds(off, L)]])
                o1_v[pl.ds(off, L)] = gv
                o2_v[pl.ds(off, L)] = gv - cv
                return 0

            lax.fori_loop(0, NSAMPLE, k_body, 0)
            return 0

        lax.fori_loop(0, NPOINT // L, s_body, 0)
        pltpu.sync_copy(o1_v, out_hbm.at[b, c])
        pltpu.sync_copy(o2_v, out_hbm.at[b, c + CIN])
        return 0

    lax.fori_loop(0, CIN // 4, chan_body, 0)


def kernel(xyz, points):
    xyz_t = jnp.transpose(xyz, (0, 2, 1))
    fps_idx, idx = _fps_and_ballquery(xyz_t)
    idx_flat = jnp.transpose(idx, (0, 2, 1)).reshape(B, _KS)
    out = _sc_gather(points, idx_flat, fps_idx)
    return out.reshape(B, 2 * CIN, NSAMPLE, NPOINT)


# speedup vs baseline: 7.7423x; 4.1159x over previous
import functools

import jax
import jax.numpy as jnp
from jax import lax
from jax.experimental import pallas as pl
from jax.experimental.pallas import tpu as pltpu
from jax.experimental.pallas import tpu_sc as plsc

NPOINT = 512
RADIUS = 0.2
NSAMPLE = 32
B, CIN, N = 8, 128, 4096
L = 16


def _fps_tc_body(xyz_ref, fidx_ref):
    x = xyz_ref[:, 0, :]
    y = xyz_ref[:, 1, :]
    z = xyz_ref[:, 2, :]
    iota = lax.broadcasted_iota(jnp.int32, (B, N), 1)
    iota_s = lax.broadcasted_iota(jnp.int32, (B, NPOINT), 1)

    def step(t, carry):
        distance, farthest = carry
        fb = lax.broadcast_in_dim(farthest, (B, NPOINT), (0, 1))
        oh_s = (iota_s == t).astype(jnp.int32)
        fidx_ref[...] = fidx_ref[...] * (1 - oh_s) + fb * oh_s
        onehot = iota == farthest
        cx = jnp.sum(jnp.where(onehot, x, 0.0), axis=1, keepdims=True)
        cy = jnp.sum(jnp.where(onehot, y, 0.0), axis=1, keepdims=True)
        cz = jnp.sum(jnp.where(onehot, z, 0.0), axis=1, keepdims=True)
        dx = x - cx
        dy = y - cy
        dz = z - cz
        dist = (dx * dx + dy * dy) + dz * dz
        distance = jnp.where(dist < distance, dist, distance)
        m = jnp.max(distance, axis=1, keepdims=True)
        farthest = jnp.min(jnp.where(distance == m, iota, N), axis=1,
                           keepdims=True).astype(jnp.int32)
        return distance, farthest

    distance0 = jnp.full((B, N), 1e10, dtype=jnp.float32)
    farthest0 = jnp.zeros((B, 1), dtype=jnp.int32)
    lax.fori_loop(0, NPOINT, step, (distance0, farthest0))


def _fps_tc(xyz):
    return pl.pallas_call(
        _fps_tc_body,
        out_shape=jax.ShapeDtypeStruct((B, NPOINT), jnp.int32),
    )(xyz)


def _fps_and_ballquery(xyz, xyz_t):
    Bb, Nn, C = xyz_t.shape
    batch_idx = jnp.arange(Bb)
    fps_idx = _fps_tc(xyz)

    new_xyz = xyz_t[batch_idx[:, None], fps_idx]
    sq = -2 * jnp.matmul(new_xyz, jnp.transpose(xyz_t, (0, 2, 1)))
    sq = sq + jnp.sum(new_xyz ** 2, -1).reshape(Bb, NPOINT, 1)
    sq = sq + jnp.sum(xyz_t ** 2, -1).reshape(Bb, 1, Nn)
    mask = ~(sq > RADIUS ** 2)
    cnt = jnp.cumsum(mask.astype(jnp.int32), axis=-1)
    cntc = jnp.minimum(cnt, NSAMPLE + 2)
    idx = jnp.stack([(cntc <= k).sum(-1) for k in range(NSAMPLE)], axis=-1)
    first = idx[:, :, 0:1]
    idx = jnp.where(idx >= Nn, first, idx)
    idx = jnp.minimum(idx, Nn - 1).astype(jnp.int32)
    return fps_idx, idx


_KS = NSAMPLE * NPOINT


@functools.cache
def _sc_gather_kernel():
    mesh = plsc.VectorSubcoreMesh(core_axis_name="c", subcore_axis_name="s")
    return pl.kernel(
        _sc_gather_body,
        mesh=mesh,
        out_type=jax.ShapeDtypeStruct((B, 2 * CIN, _KS), jnp.float32),
        compiler_params=pltpu.CompilerParams(needs_layout_passes=False),
        scratch_types=[
            pltpu.VMEM((_KS,), jnp.int32),
            pltpu.VMEM((NPOINT,), jnp.int32),
            pltpu.VMEM((N,), jnp.float32),
            pltpu.VMEM((NPOINT,), jnp.float32),
            pltpu.VMEM((_KS,), jnp.float32),
            pltpu.VMEM((_KS,), jnp.float32),
        ],
    )


def _sc_gather_body(points_hbm, idx_hbm, fps_hbm, out_hbm,
                    idx_v, fps_v, tab_v, ctr_v, o1_v, o2_v):
    wid = lax.axis_index("s") * 2 + lax.axis_index("c")
    b = wid // 4
    g = wid % 4
    pltpu.sync_copy(idx_hbm.at[b], idx_v)
    pltpu.sync_copy(fps_hbm.at[b], fps_v)

    def chan_body(ci, _):
        c = g * (CIN // 4) + ci
        pltpu.sync_copy(points_hbm.at[b, c], tab_v)

        def ctr_body(i, _):
            iv = fps_v[pl.ds(i * L, L)]
            ctr_v[pl.ds(i * L, L)] = plsc.load_gather(tab_v, [iv])
            return 0

        lax.fori_loop(0, NPOINT // L, ctr_body, 0)

        def s_body(si, _):
            cv = ctr_v[pl.ds(si * L, L)]

            def k_body(k, _):
                off = k * NPOINT + si * L
                gv = plsc.load_gather(tab_v, [idx_v[pl.ds(off, L)]])
                o1_v[pl.ds(off, L)] = gv
                o2_v[pl.ds(off, L)] = gv - cv
                return 0

            lax.fori_loop(0, NSAMPLE, k_body, 0)
            return 0

        lax.fori_loop(0, NPOINT // L, s_body, 0)
        pltpu.sync_copy(o1_v, out_hbm.at[b, c])
        pltpu.sync_copy(o2_v, out_hbm.at[b, c + CIN])
        return 0

    lax.fori_loop(0, CIN // 4, chan_body, 0)


def kernel(xyz, points):
    xyz_t = jnp.transpose(xyz, (0, 2, 1))
    fps_idx, idx = _fps_and_ballquery(xyz, xyz_t)
    idx_flat = jnp.transpose(idx, (0, 2, 1)).reshape(B, _KS)
    out = _sc_gather_kernel()(points, idx_flat, fps_idx)
    return out.reshape(B, 2 * CIN, NSAMPLE, NPOINT)


# speedup vs baseline: 9.6352x; 1.2445x over previous
import functools

import jax
import jax.numpy as jnp
from jax import lax
from jax.experimental import pallas as pl
from jax.experimental.pallas import tpu as pltpu
from jax.experimental.pallas import tpu_sc as plsc

NPOINT = 512
RADIUS = 0.2
NSAMPLE = 32
B, CIN, N = 8, 128, 4096
L = 16


def _fps_tc_body(xyz_ref, fidx_ref):
    x = xyz_ref[:, 0, :]
    y = xyz_ref[:, 1, :]
    z = xyz_ref[:, 2, :]
    iota = lax.broadcasted_iota(jnp.int32, (B, N), 1)
    iota_s = lax.broadcasted_iota(jnp.int32, (B, NPOINT), 1)

    def step(t, carry):
        distance, farthest = carry
        fb = lax.broadcast_in_dim(farthest, (B, NPOINT), (0, 1))
        oh_s = (iota_s == t).astype(jnp.int32)
        fidx_ref[...] = fidx_ref[...] * (1 - oh_s) + fb * oh_s
        onehot = iota == farthest
        cx = jnp.sum(jnp.where(onehot, x, 0.0), axis=1, keepdims=True)
        cy = jnp.sum(jnp.where(onehot, y, 0.0), axis=1, keepdims=True)
        cz = jnp.sum(jnp.where(onehot, z, 0.0), axis=1, keepdims=True)
        dx = x - cx
        dy = y - cy
        dz = z - cz
        dist = (dx * dx + dy * dy) + dz * dz
        distance = jnp.where(dist < distance, dist, distance)
        m = jnp.max(distance, axis=1, keepdims=True)
        farthest = jnp.min(jnp.where(distance == m, iota, N), axis=1,
                           keepdims=True).astype(jnp.int32)
        return distance, farthest

    distance0 = jnp.full((B, N), 1e10, dtype=jnp.float32)
    farthest0 = jnp.zeros((B, 1), dtype=jnp.int32)
    lax.fori_loop(0, NPOINT, step, (distance0, farthest0))


def _fps_tc(xyz):
    return pl.pallas_call(
        _fps_tc_body,
        out_shape=jax.ShapeDtypeStruct((B, NPOINT), jnp.int32),
    )(xyz)


_THR = RADIUS ** 2


def _ballq_tc_body(xyz_ref, new_ref, idx_ref):
    xb = xyz_ref[0]
    nb = new_ref[0]
    prod = jnp.dot(nb, xb, preferred_element_type=jnp.float32)
    cx = new_ref[0, :, 0:1]
    cy = new_ref[0, :, 1:2]
    cz = new_ref[0, :, 2:3]
    x = xyz_ref[0, 0:1, :]
    y = xyz_ref[0, 1:2, :]
    z = xyz_ref[0, 2:3, :]
    snorm = (cx * cx + cy * cy) + cz * cz
    dnorm = (x * x + y * y) + z * z
    sq = -2 * prod
    sq = sq + snorm
    sq = sq + dnorm
    mask = jnp.logical_not(sq > _THR).astype(jnp.int32)
    cnt = mask
    for sh in (1, 2, 4, 8, 16, 32, 64, 128, 256, 512, 1024, 2048):
        cnt = cnt + jnp.concatenate(
            [jnp.zeros((_SB, sh), jnp.int32), cnt[:, : N - sh]], axis=1)
    cnt = jnp.minimum(cnt, NSAMPLE + 2)
    cols = [jnp.sum((cnt <= k).astype(jnp.int32), axis=1, keepdims=True)
            for k in range(NSAMPLE)]
    idx = jnp.concatenate(cols, axis=1)
    first = idx[:, 0:1]
    idx = jnp.where(idx >= N, lax.broadcast_in_dim(first, (_SB, NSAMPLE), (0, 1)), idx)
    idx = jnp.minimum(idx, N - 1)
    idx_ref[0] = idx


_SB = 128


def _ballq_tc(xyz, new_t):
    return pl.pallas_call(
        _ballq_tc_body,
        grid=(B, NPOINT // _SB),
        in_specs=[
            pl.BlockSpec((1, 3, N), lambda b, s: (b, 0, 0)),
            pl.BlockSpec((1, _SB, 3), lambda b, s: (b, s, 0)),
        ],
        out_specs=pl.BlockSpec((1, _SB, NSAMPLE), lambda b, s: (b, s, 0)),
        out_shape=jax.ShapeDtypeStruct((B, NPOINT, NSAMPLE), jnp.int32),
    )(xyz, new_t)


def _fps_and_ballquery(xyz):
    fps_idx = _fps_tc(xyz)
    new_xyz = jnp.take_along_axis(xyz, fps_idx[:, None, :], axis=2)
    new_xyz = jnp.transpose(new_xyz, (0, 2, 1))
    idx = _ballq_tc(xyz, new_xyz)
    return fps_idx, idx


_KS = NSAMPLE * NPOINT


@functools.cache
def _sc_gather_kernel():
    mesh = plsc.VectorSubcoreMesh(core_axis_name="c", subcore_axis_name="s")
    return pl.kernel(
        _sc_gather_body,
        mesh=mesh,
        out_type=jax.ShapeDtypeStruct((B, 2 * CIN, _KS), jnp.float32),
        compiler_params=pltpu.CompilerParams(needs_layout_passes=False),
        scratch_types=[
            pltpu.VMEM((_KS,), jnp.int32),
            pltpu.VMEM((NPOINT,), jnp.int32),
            pltpu.VMEM((N,), jnp.float32),
            pltpu.VMEM((NPOINT,), jnp.float32),
            pltpu.VMEM((_KS,), jnp.float32),
            pltpu.VMEM((_KS,), jnp.float32),
        ],
    )


def _sc_gather_body(points_hbm, idx_hbm, fps_hbm, out_hbm,
                    idx_v, fps_v, tab_v, ctr_v, o1_v, o2_v):
    wid = lax.axis_index("s") * 2 + lax.axis_index("c")
    b = wid // 4
    g = wid % 4
    pltpu.sync_copy(idx_hbm.at[b], idx_v)
    pltpu.sync_copy(fps_hbm.at[b], fps_v)

    def chan_body(ci, _):
        c = g * (CIN // 4) + ci
        pltpu.sync_copy(points_hbm.at[b, c], tab_v)

        def ctr_body(i, _):
            iv = fps_v[pl.ds(i * L, L)]
            ctr_v[pl.ds(i * L, L)] = plsc.load_gather(tab_v, [iv])
            return 0

        lax.fori_loop(0, NPOINT // L, ctr_body, 0)

        def s_body(si, _):
            cv = ctr_v[pl.ds(si * L, L)]

            def k_body(k, _):
                off = k * NPOINT + si * L
                gv = plsc.load_gather(tab_v, [idx_v[pl.ds(off, L)]])
                o1_v[pl.ds(off, L)] = gv
                o2_v[pl.ds(off, L)] = gv - cv
                return 0

            lax.fori_loop(0, NSAMPLE, k_body, 0)
            return 0

        lax.fori_loop(0, NPOINT // L, s_body, 0)
        pltpu.sync_copy(o1_v, out_hbm.at[b, c])
        pltpu.sync_copy(o2_v, out_hbm.at[b, c + CIN])
        return 0

    lax.fori_loop(0, CIN // 4, chan_body, 0)


def kernel(xyz, points):
    fps_idx, idx = _fps_and_ballquery(xyz)
    idx_flat = jnp.transpose(idx, (0, 2, 1)).reshape(B, _KS)
    out = _sc_gather_kernel()(points, idx_flat, fps_idx)
    return out.reshape(B, 2 * CIN, NSAMPLE, NPOINT)


# speedup vs baseline: 11.7515x; 1.2196x over previous
import functools

import jax
import jax.numpy as jnp
from jax import lax
from jax.experimental import pallas as pl
from jax.experimental.pallas import tpu as pltpu
from jax.experimental.pallas import tpu_sc as plsc

NPOINT = 512
RADIUS = 0.2
NSAMPLE = 32
B, CIN, N = 8, 128, 4096
L = 16


def _fps_tc_body(xyz_ref, fidx_ref):
    x = xyz_ref[:, 0, :]
    y = xyz_ref[:, 1, :]
    z = xyz_ref[:, 2, :]
    iota = lax.broadcasted_iota(jnp.int32, (B, N), 1)
    iota_s = lax.broadcasted_iota(jnp.int32, (B, NPOINT), 1)

    def step(t, carry):
        distance, farthest = carry
        fb = lax.broadcast_in_dim(farthest, (B, NPOINT), (0, 1))
        oh_s = (iota_s == t).astype(jnp.int32)
        fidx_ref[...] = fidx_ref[...] * (1 - oh_s) + fb * oh_s
        onehot = iota == farthest
        cx = jnp.sum(jnp.where(onehot, x, 0.0), axis=1, keepdims=True)
        cy = jnp.sum(jnp.where(onehot, y, 0.0), axis=1, keepdims=True)
        cz = jnp.sum(jnp.where(onehot, z, 0.0), axis=1, keepdims=True)
        dx = x - cx
        dy = y - cy
        dz = z - cz
        dist = (dx * dx + dy * dy) + dz * dz
        distance = jnp.where(dist < distance, dist, distance)
        m = jnp.max(distance, axis=1, keepdims=True)
        farthest = jnp.min(jnp.where(distance == m, iota, N), axis=1,
                           keepdims=True).astype(jnp.int32)
        return distance, farthest

    distance0 = jnp.full((B, N), 1e10, dtype=jnp.float32)
    farthest0 = jnp.zeros((B, 1), dtype=jnp.int32)
    lax.fori_loop(0, NPOINT, step, (distance0, farthest0))


def _fps_tc(xyz):
    return pl.pallas_call(
        _fps_tc_body,
        out_shape=jax.ShapeDtypeStruct((B, NPOINT), jnp.int32),
    )(xyz)


_THR = RADIUS ** 2


def _ballq_tc_body(xyz_ref, new_ref, idx_ref):
    xb = xyz_ref[0]
    nb = new_ref[0]
    prod = jnp.dot(nb, xb, preferred_element_type=jnp.float32)
    cx = new_ref[0, :, 0:1]
    cy = new_ref[0, :, 1:2]
    cz = new_ref[0, :, 2:3]
    x = xyz_ref[0, 0:1, :]
    y = xyz_ref[0, 1:2, :]
    z = xyz_ref[0, 2:3, :]
    snorm = (cx * cx + cy * cy) + cz * cz
    dnorm = (x * x + y * y) + z * z
    sq = -2 * prod
    sq = sq + snorm
    sq = sq + dnorm
    mask = jnp.logical_not(sq > _THR).astype(jnp.int32)
    cnt = mask
    for sh in (1, 2, 4, 8, 16, 32, 64, 128, 256, 512, 1024, 2048):
        cnt = cnt + jnp.concatenate(
            [jnp.zeros((_SB, sh), jnp.int32), cnt[:, : N - sh]], axis=1)
    cnt = jnp.minimum(cnt, NSAMPLE + 2)
    cols = [jnp.sum((cnt <= k).astype(jnp.int32), axis=1, keepdims=True)
            for k in range(NSAMPLE)]
    idx = jnp.concatenate(cols, axis=1)
    first = idx[:, 0:1]
    idx = jnp.where(idx >= N, lax.broadcast_in_dim(first, (_SB, NSAMPLE), (0, 1)), idx)
    idx = jnp.minimum(idx, N - 1)
    idx_ref[0] = jnp.transpose(idx)


_SB = 128


def _ballq_tc(xyz, new_t):
    return pl.pallas_call(
        _ballq_tc_body,
        grid=(B, NPOINT // _SB),
        in_specs=[
            pl.BlockSpec((1, 3, N), lambda b, s: (b, 0, 0)),
            pl.BlockSpec((1, _SB, 3), lambda b, s: (b, s, 0)),
        ],
        out_specs=pl.BlockSpec((1, NSAMPLE, _SB), lambda b, s: (b, 0, s)),
        out_shape=jax.ShapeDtypeStruct((B, NSAMPLE, NPOINT), jnp.int32),
    )(xyz, new_t)


def _fps_and_ballquery(xyz):
    fps_idx = _fps_tc(xyz)
    new_xyz = jnp.take_along_axis(xyz, fps_idx[:, None, :], axis=2)
    new_xyz = jnp.transpose(new_xyz, (0, 2, 1))
    idx = _ballq_tc(xyz, new_xyz)
    return fps_idx, idx


_KS = NSAMPLE * NPOINT


_TB = 4


@functools.cache
def _sc_gather_kernel():
    mesh = plsc.VectorSubcoreMesh(core_axis_name="c", subcore_axis_name="s")
    return pl.kernel(
        _sc_gather_body,
        mesh=mesh,
        out_type=jax.ShapeDtypeStruct((B, 2 * CIN, _KS), jnp.float32),
        compiler_params=pltpu.CompilerParams(needs_layout_passes=False),
        scratch_types=[
            pltpu.VMEM((_KS,), jnp.int32),
            pltpu.VMEM((NPOINT,), jnp.int32),
            pltpu.VMEM((2 * _TB * N,), jnp.float32),
            pltpu.VMEM((NPOINT,), jnp.float32),
            pltpu.VMEM((2 * _KS,), jnp.float32),
            pltpu.VMEM((2 * _KS,), jnp.float32),
            pltpu.SemaphoreType.DMA,
            pltpu.SemaphoreType.DMA,
            pltpu.SemaphoreType.DMA,
            pltpu.SemaphoreType.DMA,
            pltpu.SemaphoreType.DMA,
            pltpu.SemaphoreType.DMA,
        ],
    )


def _sc_gather_body(points_hbm, idx_hbm, fps_hbm, out_hbm,
                    idx_v, fps_v, tab_v, ctr_v, o1_v, o2_v,
                    sem_in0, sem_in1, sem_o1a, sem_o1b, sem_o2a, sem_o2b):
    wid = lax.axis_index("s") * 2 + lax.axis_index("c")
    b = wid // 4
    g = wid % 4
    c0 = g * (CIN // 4)
    pltpu.sync_copy(idx_hbm.at[b], idx_v)
    pltpu.sync_copy(fps_hbm.at[b], fps_v)

    nblk = (CIN // 4) // _TB
    sem_in = (sem_in0, sem_in1)
    sem_o1 = (sem_o1a, sem_o1b)
    sem_o2 = (sem_o2a, sem_o2b)
    h_in = [None] * nblk
    h_o1 = {}
    h_o2 = {}
    def _start_block(bb2, dbuf):
        return [pltpu.async_copy(
                    points_hbm.at[b, c0 + bb2 * _TB + u2],
                    tab_v.at[pl.ds((dbuf * _TB + u2) * N, N)], sem_in[dbuf])
                for u2 in range(_TB)]

    h_in[0] = _start_block(0, 0)
    for bb in range(nblk):
        buf = bb % 2
        if bb + 1 < nblk:
            h_in[bb + 1] = _start_block(bb + 1, 1 - buf)
        for h in h_in[bb]:
            h.wait()
        for u in range(_TB):
            j = bb * _TB + u
            obuf = j % 2
            c = c0 + j
            obase = obuf * _KS
            if j >= 2:
                h_o1[j - 2].wait()
                h_o2[j - 2].wait()
            tab = tab_v.at[pl.ds((buf * _TB + u) * N, N)]

            def ctr_body(i, _):
                iv = fps_v[pl.ds(i * L, L)]
                ctr_v[pl.ds(i * L, L)] = plsc.load_gather(tab, [iv])
                return 0

            lax.fori_loop(0, NPOINT // L, ctr_body, 0)

            def grp_body(i, _):
                k = i // 4
                sb = (i % 4) * 8
                for uu in range(8):
                    s16 = (sb + uu) * L
                    off = k * NPOINT + s16
                    gv = plsc.load_gather(tab, [idx_v[pl.ds(off, L)]])
                    cv = ctr_v[pl.ds(s16, L)]
                    o1_v[pl.ds(obase + off, L)] = gv
                    o2_v[pl.ds(obase + off, L)] = gv - cv
                return 0

            lax.fori_loop(0, _KS // (8 * L), grp_body, 0)
            h_o1[j] = pltpu.async_copy(
                o1_v.at[pl.ds(obase, _KS)], out_hbm.at[b, c], sem_o1[obuf])
            h_o2[j] = pltpu.async_copy(
                o2_v.at[pl.ds(obase, _KS)], out_hbm.at[b, c + CIN], sem_o2[obuf])
    h_o1[CIN // 4 - 2].wait()
    h_o2[CIN // 4 - 2].wait()
    h_o1[CIN // 4 - 1].wait()
    h_o2[CIN // 4 - 1].wait()


def kernel(xyz, points):
    fps_idx, idx = _fps_and_ballquery(xyz)
    idx_flat = idx.reshape(B, _KS)
    out = _sc_gather_kernel()(points, idx_flat, fps_idx)
    return out.reshape(B, 2 * CIN, NSAMPLE, NPOINT)
